# baseline (speedup 1.0000x reference)
import functools

import jax
import jax.numpy as jnp
from jax import lax
from jax.experimental import pallas as pl
from jax.experimental.pallas import tpu as pltpu
from jax.experimental.pallas import tpu_sc as plsc

B = 1024
CTX = 20
D = 64
VOCAB = 100000

NC = 2
NS = 16
NW = NC * NS
IDX_PER_W = B * CTX // NW
ROWS_PER_W = B // NW
GCHUNK = 128
NCHUNK = IDX_PER_W // GCHUNK


def _sc_gather_sum(idx_ref, table_ref, out_ref, idx_v, rows_v, acc_v, sem):
    wid = lax.axis_index("s") * NC + lax.axis_index("c")
    pltpu.sync_copy(idx_ref.at[wid], idx_v)
    copies = [
        pltpu.async_copy(
            table_ref.at[idx_v.at[j]],
            rows_v.at[pl.ds(j * GCHUNK, GCHUNK)],
            sem,
        )
        for j in range(NCHUNK)
    ]
    for c in copies:
        c.wait()

    def body(r, _):
        for l in range(D // 16):
            acc = rows_v[r * CTX, pl.ds(l * 16, 16)]
            for c in range(1, CTX):
                acc = acc + rows_v[r * CTX + c, pl.ds(l * 16, 16)]
            acc_v[r, pl.ds(l * 16, 16)] = acc
        return 0

    lax.fori_loop(0, ROWS_PER_W, body, 0)
    pltpu.sync_copy(acc_v, out_ref.at[pl.ds(wid * ROWS_PER_W, ROWS_PER_W)])


@functools.partial(
    pl.kernel,
    out_type=jax.ShapeDtypeStruct((B, D), jnp.float32),
    mesh=plsc.VectorSubcoreMesh(core_axis_name="c", subcore_axis_name="s"),
    scratch_types=[
        pltpu.VMEM((NCHUNK, GCHUNK), jnp.int32),
        pltpu.VMEM((IDX_PER_W, D), jnp.float32),
        pltpu.VMEM((ROWS_PER_W, D), jnp.float32),
        pltpu.SemaphoreType.DMA,
    ],
    compiler_params=pltpu.CompilerParams(use_tc_tiling_on_sc=False),
)
def _gather_sum(idx, table, out, idx_v, rows_v, acc_v, sem):
    _sc_gather_sum(idx, table, out, idx_v, rows_v, acc_v, sem)


TV = 4096
NV = (VOCAB + TV - 1) // TV
VP = NV * TV
NBB = 1
BB = B // NBB


def _mm_lse_kernel(wt_ref, se0_ref, se1_ref, out_ref, s0_ref, s1_ref):
    i = pl.program_id(0)
    v = pl.program_id(1)
    par = lax.rem(i, 2)

    @pl.when(i < NBB)
    def _phase0():
        x0 = lax.dot_general(
            wt_ref[...], se0_ref[...],
            (((0,), (1,)), ((), ())),
            preferred_element_type=jnp.float32,
        )
        red = jnp.sum(jnp.exp(x0), axis=0, keepdims=True)
        old = jnp.where(par == 0, s0_ref[...], s1_ref[...])
        s_new = red + jnp.where(v == 0, jnp.zeros_like(old), old)

        @pl.when(par == 0)
        def _():
            s0_ref[...] = s_new

        @pl.when(par == 1)
        def _():
            s1_ref[...] = s_new

    @pl.when(i > 0)
    def _phase1():
        x1 = lax.dot_general(
            wt_ref[...], se1_ref[...],
            (((0,), (1,)), ((), ())),
            preferred_element_type=jnp.float32,
        )
        logs = jnp.log(jnp.where(par == 1, s0_ref[...], s1_ref[...]))
        out_ref[...] = x1 - logs


def _mm_log_softmax(wt_aug, se_aug):
    return pl.pallas_call(
        _mm_lse_kernel,
        grid=(NBB + 1, NV),
        in_specs=[
            pl.BlockSpec((D + 1, TV), lambda i, v: (0, v)),
            pl.BlockSpec((BB, D + 1), lambda i, v: (jnp.minimum(i, NBB - 1), 0)),
            pl.BlockSpec((BB, D + 1), lambda i, v: (jnp.maximum(i - 1, 0), 0)),
        ],
        out_specs=pl.BlockSpec(
            (TV, BB),
            lambda i, v: (jnp.where(i == 0, 0, v), jnp.maximum(i - 1, 0)),
        ),
        out_shape=jax.ShapeDtypeStruct((VOCAB, B), jnp.float32),
        scratch_shapes=[
            pltpu.VMEM((1, BB), jnp.float32),
            pltpu.VMEM((1, BB), jnp.float32),
        ],
        compiler_params=pltpu.CompilerParams(
            dimension_semantics=("arbitrary", "arbitrary"),
        ),
    )(wt_aug, se_aug, se_aug)


def kernel(context, emb_table, W, b):
    idx = context.reshape(NW, NCHUNK, GCHUNK).astype(jnp.int32)
    sum_embeds = _gather_sum(idx, emb_table)

    wt = jnp.pad(W.T.astype(jnp.bfloat16), ((0, 0), (0, VP - VOCAB)))
    b_row = jnp.pad(b.astype(jnp.bfloat16), (0, VP - VOCAB),
                    constant_values=jnp.bfloat16(-30000.0))
    wt_aug = jnp.concatenate([wt, b_row[None, :]], axis=0)
    se_aug = jnp.concatenate(
        [sum_embeds.astype(jnp.bfloat16), jnp.ones((B, 1), jnp.bfloat16)],
        axis=1)

    out_t = _mm_log_softmax(wt_aug, se_aug)
    return out_t.T


# speedup vs baseline: 1.0552x; 1.0552x over previous
import functools

import jax
import jax.numpy as jnp
from jax import lax
from jax.experimental import pallas as pl
from jax.experimental.pallas import tpu as pltpu
from jax.experimental.pallas import tpu_sc as plsc

B = 1024
CTX = 20
D = 64
VOCAB = 100000

NC = 2
NS = 16
NW = NC * NS
IDX_PER_W = B * CTX // NW
ROWS_PER_W = B // NW
GCHUNK = 128
NCHUNK = IDX_PER_W // GCHUNK


def _sc_gather_sum(idx_ref, table_ref, out_ref, idx_v, rows_v, acc_v, sem):
    wid = lax.axis_index("s") * NC + lax.axis_index("c")
    pltpu.sync_copy(idx_ref.at[wid], idx_v)
    copies = [
        pltpu.async_copy(
            table_ref.at[idx_v.at[j]],
            rows_v.at[pl.ds(j * GCHUNK, GCHUNK)],
            sem,
        )
        for j in range(NCHUNK)
    ]
    for c in copies:
        c.wait()

    def body(r, _):
        for l in range(D // 16):
            acc = rows_v[r * CTX, pl.ds(l * 16, 16)]
            for c in range(1, CTX):
                acc = acc + rows_v[r * CTX + c, pl.ds(l * 16, 16)]
            acc_v[r, pl.ds(l * 16, 16)] = acc
        return 0

    lax.fori_loop(0, ROWS_PER_W, body, 0)
    pltpu.sync_copy(acc_v, out_ref.at[pl.ds(wid * ROWS_PER_W, ROWS_PER_W)])


@functools.partial(
    pl.kernel,
    out_type=jax.ShapeDtypeStruct((B, D), jnp.float32),
    mesh=plsc.VectorSubcoreMesh(core_axis_name="c", subcore_axis_name="s"),
    scratch_types=[
        pltpu.VMEM((NCHUNK, GCHUNK), jnp.int32),
        pltpu.VMEM((IDX_PER_W, D), jnp.float32),
        pltpu.VMEM((ROWS_PER_W, D), jnp.float32),
        pltpu.SemaphoreType.DMA,
    ],
    compiler_params=pltpu.CompilerParams(use_tc_tiling_on_sc=False),
)
def _gather_sum(idx, table, out, idx_v, rows_v, acc_v, sem):
    _sc_gather_sum(idx, table, out, idx_v, rows_v, acc_v, sem)


TV = 8192
NV = (VOCAB + TV - 1) // TV
VP = NV * TV
NBB = 2
BB = B // NBB


def _mm_lse_kernel(wt_ref, se0_ref, se1_ref, out_ref, s0_ref, s1_ref):
    i = pl.program_id(0)
    v = pl.program_id(1)
    par = lax.rem(i, 2)

    @pl.when(i < NBB)
    def _phase0():
        x0 = lax.dot_general(
            wt_ref[...], se0_ref[...],
            (((0,), (1,)), ((), ())),
            preferred_element_type=jnp.float32,
        )
        red = jnp.sum(jnp.exp(x0), axis=0, keepdims=True)
        old = jnp.where(par == 0, s0_ref[...], s1_ref[...])
        s_new = red + jnp.where(v == 0, jnp.zeros_like(old), old)

        @pl.when(par == 0)
        def _():
            s0_ref[...] = s_new

        @pl.when(par == 1)
        def _():
            s1_ref[...] = s_new

    @pl.when(i > 0)
    def _phase1():
        x1 = lax.dot_general(
            wt_ref[...], se1_ref[...],
            (((0,), (1,)), ((), ())),
            preferred_element_type=jnp.float32,
        )
        logs = jnp.log(jnp.where(par == 1, s0_ref[...], s1_ref[...]))
        out_ref[...] = x1 - logs


def _mm_log_softmax(wt_aug, se_aug):
    return pl.pallas_call(
        _mm_lse_kernel,
        grid=(NBB + 1, NV),
        in_specs=[
            pl.BlockSpec((D + 1, TV), lambda i, v: (0, v)),
            pl.BlockSpec((BB, D + 1), lambda i, v: (jnp.minimum(i, NBB - 1), 0)),
            pl.BlockSpec((BB, D + 1), lambda i, v: (jnp.maximum(i - 1, 0), 0)),
        ],
        out_specs=pl.BlockSpec(
            (TV, BB),
            lambda i, v: (jnp.where(i == 0, 0, v), jnp.maximum(i - 1, 0)),
        ),
        out_shape=jax.ShapeDtypeStruct((VOCAB, B), jnp.float32),
        scratch_shapes=[
            pltpu.VMEM((1, BB), jnp.float32),
            pltpu.VMEM((1, BB), jnp.float32),
        ],
        compiler_params=pltpu.CompilerParams(
            dimension_semantics=("arbitrary", "arbitrary"),
        ),
    )(wt_aug, se_aug, se_aug)


def kernel(context, emb_table, W, b):
    idx = context.reshape(NW, NCHUNK, GCHUNK).astype(jnp.int32)
    sum_embeds = _gather_sum(idx, emb_table)

    wt = jnp.pad(W.T.astype(jnp.bfloat16), ((0, 0), (0, VP - VOCAB)))
    b_row = jnp.pad(b.astype(jnp.bfloat16), (0, VP - VOCAB),
                    constant_values=jnp.bfloat16(-30000.0))
    wt_aug = jnp.concatenate([wt, b_row[None, :]], axis=0)
    se_aug = jnp.concatenate(
        [sum_embeds.astype(jnp.bfloat16), jnp.ones((B, 1), jnp.bfloat16)],
        axis=1)

    out_t = _mm_log_softmax(wt_aug, se_aug)
    return out_t.T


# speedup vs baseline: 1.1174x; 1.0590x over previous
import functools

import jax
import jax.numpy as jnp
from jax import lax
from jax.experimental import pallas as pl
from jax.experimental.pallas import tpu as pltpu
from jax.experimental.pallas import tpu_sc as plsc

B = 1024
CTX = 20
D = 64
VOCAB = 100000

NC = 2
NS = 16
NW = NC * NS
IDX_PER_W = B * CTX // NW
ROWS_PER_W = B // NW
GCHUNK = 128
NCHUNK = IDX_PER_W // GCHUNK


def _sc_gather_sum(idx_ref, table_ref, out_ref, idx_v, rows_v, acc_v, sem):
    wid = lax.axis_index("s") * NC + lax.axis_index("c")
    pltpu.sync_copy(idx_ref.at[wid], idx_v)
    copies = [
        pltpu.async_copy(
            table_ref.at[idx_v.at[j]],
            rows_v.at[pl.ds(j * GCHUNK, GCHUNK)],
            sem,
        )
        for j in range(NCHUNK)
    ]
    for c in copies:
        c.wait()

    def body(r, _):
        for l in range(D // 16):
            acc = rows_v[r * CTX, pl.ds(l * 16, 16)]
            for c in range(1, CTX):
                acc = acc + rows_v[r * CTX + c, pl.ds(l * 16, 16)]
            acc_v[r, pl.ds(l * 16, 16)] = acc
        return 0

    lax.fori_loop(0, ROWS_PER_W, body, 0)
    pltpu.sync_copy(acc_v, out_ref.at[pl.ds(wid * ROWS_PER_W, ROWS_PER_W)])


@functools.partial(
    pl.kernel,
    out_type=jax.ShapeDtypeStruct((B, D), jnp.float32),
    mesh=plsc.VectorSubcoreMesh(core_axis_name="c", subcore_axis_name="s"),
    scratch_types=[
        pltpu.VMEM((NCHUNK, GCHUNK), jnp.int32),
        pltpu.VMEM((IDX_PER_W, D), jnp.float32),
        pltpu.VMEM((ROWS_PER_W, D), jnp.float32),
        pltpu.SemaphoreType.DMA,
    ],
    compiler_params=pltpu.CompilerParams(use_tc_tiling_on_sc=False),
)
def _gather_sum(idx, table, out, idx_v, rows_v, acc_v, sem):
    _sc_gather_sum(idx, table, out, idx_v, rows_v, acc_v, sem)


SPLIT = 50176
TROWS = 2 * SPLIT
NRB = SPLIT // 1024


def _relayout_kernel(lo_ref, hi_ref, out_ref):
    out_ref[:, 0:64] = lo_ref[...].T
    out_ref[:, 64:128] = hi_ref[...].T


def _relayout_table(table_t):
    return pl.pallas_call(
        _relayout_kernel,
        grid=(NRB,),
        in_specs=[
            pl.BlockSpec((D, 1024), lambda v: (0, v)),
            pl.BlockSpec((D, 1024), lambda v: (0, NRB + v)),
        ],
        out_specs=pl.BlockSpec((1024, 128), lambda v: (v, 0)),
        out_shape=jax.ShapeDtypeStruct((SPLIT, 128), jnp.float32),
    )(table_t, table_t)


TV = 8192
NV = (VOCAB + TV - 1) // TV
VP = NV * TV
NBB = 2
BB = B // NBB


def _mm_lse_kernel(wt_ref, se0_ref, se1_ref, out_ref, s0_ref, s1_ref):
    i = pl.program_id(0)
    v = pl.program_id(1)
    par = lax.rem(i, 2)

    @pl.when(i < NBB)
    def _phase0():
        x0 = lax.dot_general(
            wt_ref[...], se0_ref[...],
            (((0,), (1,)), ((), ())),
            preferred_element_type=jnp.float32,
        )
        red = jnp.sum(jnp.exp(x0), axis=0, keepdims=True)
        old = jnp.where(par == 0, s0_ref[...], s1_ref[...])
        s_new = red + jnp.where(v == 0, jnp.zeros_like(old), old)

        @pl.when(par == 0)
        def _():
            s0_ref[...] = s_new

        @pl.when(par == 1)
        def _():
            s1_ref[...] = s_new

    @pl.when(i > 0)
    def _phase1():
        x1 = lax.dot_general(
            wt_ref[...], se1_ref[...],
            (((0,), (1,)), ((), ())),
            preferred_element_type=jnp.float32,
        )
        logs = jnp.log(jnp.where(par == 1, s0_ref[...], s1_ref[...]))
        out_ref[...] = x1 - logs


def _mm_log_softmax(wt_aug, se_aug):
    return pl.pallas_call(
        _mm_lse_kernel,
        grid=(NBB + 1, NV),
        in_specs=[
            pl.BlockSpec((D + 1, TV), lambda i, v: (0, v)),
            pl.BlockSpec((BB, D + 1), lambda i, v: (jnp.minimum(i, NBB - 1), 0)),
            pl.BlockSpec((BB, D + 1), lambda i, v: (jnp.maximum(i - 1, 0), 0)),
        ],
        out_specs=pl.BlockSpec(
            (TV, BB),
            lambda i, v: (jnp.where(i == 0, 0, v), jnp.maximum(i - 1, 0)),
        ),
        out_shape=jax.ShapeDtypeStruct((VOCAB, B), jnp.float32),
        scratch_shapes=[
            pltpu.VMEM((1, BB), jnp.float32),
            pltpu.VMEM((1, BB), jnp.float32),
        ],
        compiler_params=pltpu.CompilerParams(
            dimension_semantics=("arbitrary", "arbitrary"),
        ),
    )(wt_aug, se_aug, se_aug)


def kernel(context, emb_table, W, b):
    table_lin = _relayout_table(emb_table.T).reshape(TROWS, D)
    ctx = context.astype(jnp.int32)
    remapped = jnp.where(ctx < SPLIT, 2 * ctx, 2 * (ctx - SPLIT) + 1)
    idx = remapped.reshape(NW, NCHUNK, GCHUNK)
    sum_embeds = _gather_sum(idx, table_lin)

    wt = jnp.pad(W.T.astype(jnp.bfloat16), ((0, 0), (0, VP - VOCAB)))
    b_row = jnp.pad(b.astype(jnp.bfloat16), (0, VP - VOCAB),
                    constant_values=jnp.bfloat16(-30000.0))
    wt_aug = jnp.concatenate([wt, b_row[None, :]], axis=0)
    se_aug = jnp.concatenate(
        [sum_embeds.astype(jnp.bfloat16), jnp.ones((B, 1), jnp.bfloat16)],
        axis=1)

    out_t = _mm_log_softmax(wt_aug, se_aug)
    return out_t.T


# speedup vs baseline: 1.1944x; 1.0689x over previous
import functools

import jax
import jax.numpy as jnp
from jax import lax
from jax.experimental import pallas as pl
from jax.experimental.pallas import tpu as pltpu
from jax.experimental.pallas import tpu_sc as plsc

B = 1024
CTX = 20
D = 64
VOCAB = 100000

NC = 2
NS = 16
NW = NC * NS
IDX_PER_W = B * CTX // NW
ROWS_PER_W = B // NW
GCHUNK = 128
NCHUNK = IDX_PER_W // GCHUNK


def _sc_gather_sum(idx_ref, table_ref, out_ref, idx_v, rows_v, acc_v, sem):
    wid = lax.axis_index("s") * NC + lax.axis_index("c")
    pltpu.sync_copy(idx_ref.at[wid], idx_v)
    copies = [
        pltpu.async_copy(
            table_ref.at[idx_v.at[j]],
            rows_v.at[pl.ds(j * GCHUNK, GCHUNK)],
            sem,
        )
        for j in range(NCHUNK)
    ]
    for c in copies:
        c.wait()

    def body(r, _):
        for l in range(D // 16):
            acc = rows_v[r * CTX, pl.ds(l * 16, 16)]
            for c in range(1, CTX):
                acc = acc + rows_v[r * CTX + c, pl.ds(l * 16, 16)]
            acc_v[r, pl.ds(l * 16, 16)] = acc
        return 0

    lax.fori_loop(0, ROWS_PER_W, body, 0)
    pltpu.sync_copy(acc_v, out_ref.at[pl.ds(wid * ROWS_PER_W, ROWS_PER_W)])


@functools.partial(
    pl.kernel,
    out_type=jax.ShapeDtypeStruct((B, D), jnp.float32),
    mesh=plsc.VectorSubcoreMesh(core_axis_name="c", subcore_axis_name="s"),
    scratch_types=[
        pltpu.VMEM((NCHUNK, GCHUNK), jnp.int32),
        pltpu.VMEM((IDX_PER_W, D), jnp.float32),
        pltpu.VMEM((ROWS_PER_W, D), jnp.float32),
        pltpu.SemaphoreType.DMA,
    ],
    compiler_params=pltpu.CompilerParams(use_tc_tiling_on_sc=False),
)
def _gather_sum(idx, table, out, idx_v, rows_v, acc_v, sem):
    _sc_gather_sum(idx, table, out, idx_v, rows_v, acc_v, sem)


RB = 4096
NRB = 13
SPLIT = RB * NRB
TROWS = 2 * SPLIT
_NCB = (VOCAB + RB - 1) // RB


def _relayout_kernel(lo_ref, hi_ref, out_ref):
    out_ref[:, 0:64] = lo_ref[...].T
    out_ref[:, 64:128] = hi_ref[...].T


def _relayout_table(table_t):
    return pl.pallas_call(
        _relayout_kernel,
        grid=(NRB,),
        in_specs=[
            pl.BlockSpec((D, RB), lambda v: (0, v)),
            pl.BlockSpec((D, RB), lambda v: (0, jnp.minimum(NRB + v, _NCB - 1))),
        ],
        out_specs=pl.BlockSpec((RB, 128), lambda v: (v, 0)),
        out_shape=jax.ShapeDtypeStruct((SPLIT, 128), jnp.float32),
    )(table_t, table_t)


TV = 8192
NV = (VOCAB + TV - 1) // TV
VP = NV * TV
NBB = 2
BB = B // NBB


def _mm_lse_kernel(wt_ref, se0_ref, se1_ref, out_ref, s0_ref, s1_ref):
    i = pl.program_id(0)
    v = pl.program_id(1)
    par = lax.rem(i, 2)

    @pl.when(i < NBB)
    def _phase0():
        x0 = lax.dot_general(
            wt_ref[...], se0_ref[...],
            (((0,), (1,)), ((), ())),
            preferred_element_type=jnp.float32,
        )
        red = jnp.sum(jnp.exp(x0), axis=0, keepdims=True)
        old = jnp.where(par == 0, s0_ref[...], s1_ref[...])
        s_new = red + jnp.where(v == 0, jnp.zeros_like(old), old)

        @pl.when(par == 0)
        def _():
            s0_ref[...] = s_new

        @pl.when(par == 1)
        def _():
            s1_ref[...] = s_new

    @pl.when(i > 0)
    def _phase1():
        x1 = lax.dot_general(
            wt_ref[...], se1_ref[...],
            (((0,), (1,)), ((), ())),
            preferred_element_type=jnp.float32,
        )
        logs = jnp.log(jnp.where(par == 1, s0_ref[...], s1_ref[...]))
        out_ref[...] = x1 - logs


def _mm_log_softmax(wt_aug, se_aug):
    return pl.pallas_call(
        _mm_lse_kernel,
        grid=(NBB + 1, NV),
        in_specs=[
            pl.BlockSpec((D + 1, TV), lambda i, v: (0, v)),
            pl.BlockSpec((BB, D + 1), lambda i, v: (jnp.minimum(i, NBB - 1), 0)),
            pl.BlockSpec((BB, D + 1), lambda i, v: (jnp.maximum(i - 1, 0), 0)),
        ],
        out_specs=pl.BlockSpec(
            (TV, BB),
            lambda i, v: (jnp.where(i == 0, 0, v), jnp.maximum(i - 1, 0)),
        ),
        out_shape=jax.ShapeDtypeStruct((VOCAB, B), jnp.float32),
        scratch_shapes=[
            pltpu.VMEM((1, BB), jnp.float32),
            pltpu.VMEM((1, BB), jnp.float32),
        ],
        compiler_params=pltpu.CompilerParams(
            dimension_semantics=("arbitrary", "arbitrary"),
        ),
    )(wt_aug, se_aug, se_aug)


def kernel(context, emb_table, W, b):
    table_lin = _relayout_table(emb_table.T).reshape(TROWS, D)
    ctx = context.astype(jnp.int32)
    remapped = jnp.where(ctx < SPLIT, 2 * ctx, 2 * (ctx - SPLIT) + 1)
    idx = remapped.reshape(NW, NCHUNK, GCHUNK)
    sum_embeds = _gather_sum(idx, table_lin)

    wt = jnp.pad(W.T.astype(jnp.bfloat16), ((0, 0), (0, VP - VOCAB)))
    b_row = jnp.pad(b.astype(jnp.bfloat16), (0, VP - VOCAB),
                    constant_values=jnp.bfloat16(-30000.0))
    wt_aug = jnp.concatenate([wt, b_row[None, :]], axis=0)
    se_aug = jnp.concatenate(
        [sum_embeds.astype(jnp.bfloat16), jnp.ones((B, 1), jnp.bfloat16)],
        axis=1)

    out_t = _mm_log_softmax(wt_aug, se_aug)
    return out_t.T


# speedup vs baseline: 1.1991x; 1.0039x over previous
import functools

import jax
import jax.numpy as jnp
from jax import lax
from jax.experimental import pallas as pl
from jax.experimental.pallas import tpu as pltpu
from jax.experimental.pallas import tpu_sc as plsc

B = 1024
CTX = 20
D = 64
VOCAB = 100000

NC = 2
NS = 16
NW = NC * NS
IDX_PER_W = B * CTX // NW
ROWS_PER_W = B // NW
GCHUNK = 128
NCHUNK = IDX_PER_W // GCHUNK


def _sc_gather_sum(idx_ref, table_ref, out_ref, idx_v, rows_v, acc_v, sem):
    wid = lax.axis_index("s") * NC + lax.axis_index("c")
    pltpu.sync_copy(idx_ref.at[wid], idx_v)
    copies = [
        pltpu.async_copy(
            table_ref.at[idx_v.at[j]],
            rows_v.at[pl.ds(j * GCHUNK, GCHUNK)],
            sem,
        )
        for j in range(NCHUNK)
    ]
    for c in copies:
        c.wait()

    def body(r, _):
        for l in range(D // 16):
            acc = rows_v[r * CTX, pl.ds(l * 16, 16)]
            for c in range(1, CTX):
                acc = acc + rows_v[r * CTX + c, pl.ds(l * 16, 16)]
            acc_v[r, pl.ds(l * 16, 16)] = acc
        return 0

    lax.fori_loop(0, ROWS_PER_W, body, 0)
    pltpu.sync_copy(acc_v, out_ref.at[pl.ds(wid * ROWS_PER_W, ROWS_PER_W)])


@functools.partial(
    pl.kernel,
    out_type=jax.ShapeDtypeStruct((B, D), jnp.float32),
    mesh=plsc.VectorSubcoreMesh(core_axis_name="c", subcore_axis_name="s"),
    scratch_types=[
        pltpu.VMEM((NCHUNK, GCHUNK), jnp.int32),
        pltpu.VMEM((IDX_PER_W, D), jnp.float32),
        pltpu.VMEM((ROWS_PER_W, D), jnp.float32),
        pltpu.SemaphoreType.DMA,
    ],
    compiler_params=pltpu.CompilerParams(use_tc_tiling_on_sc=False),
)
def _gather_sum(idx, table, out, idx_v, rows_v, acc_v, sem):
    _sc_gather_sum(idx, table, out, idx_v, rows_v, acc_v, sem)


RB = 8192
NRB = 7
SPLIT = RB * NRB
TROWS = 2 * SPLIT
_NCB = (VOCAB + RB - 1) // RB


def _relayout_kernel(lo_ref, hi_ref, out_ref):
    out_ref[:, 0:64] = lo_ref[...].T
    out_ref[:, 64:128] = hi_ref[...].T


def _relayout_table(table_t):
    return pl.pallas_call(
        _relayout_kernel,
        grid=(NRB,),
        in_specs=[
            pl.BlockSpec((D, RB), lambda v: (0, v)),
            pl.BlockSpec((D, RB), lambda v: (0, jnp.minimum(NRB + v, _NCB - 1))),
        ],
        out_specs=pl.BlockSpec((RB, 128), lambda v: (v, 0)),
        out_shape=jax.ShapeDtypeStruct((SPLIT, 128), jnp.float32),
    )(table_t, table_t)


TV = 8192
NV = (VOCAB + TV - 1) // TV
VP = NV * TV
NBB = 2
BB = B // NBB


def _mm_lse_kernel(wt_ref, se0_ref, se1_ref, out_ref, s0_ref, s1_ref):
    i = pl.program_id(0)
    v = pl.program_id(1)
    par = lax.rem(i, 2)

    @pl.when(i < NBB)
    def _phase0():
        x0 = lax.dot_general(
            wt_ref[...], se0_ref[...],
            (((0,), (1,)), ((), ())),
            preferred_element_type=jnp.float32,
        )
        red = jnp.sum(jnp.exp(x0), axis=0, keepdims=True)
        old = jnp.where(par == 0, s0_ref[...], s1_ref[...])
        s_new = red + jnp.where(v == 0, jnp.zeros_like(old), old)

        @pl.when(par == 0)
        def _():
            s0_ref[...] = s_new

        @pl.when(par == 1)
        def _():
            s1_ref[...] = s_new

    @pl.when(i > 0)
    def _phase1():
        x1 = lax.dot_general(
            wt_ref[...], se1_ref[...],
            (((0,), (1,)), ((), ())),
            preferred_element_type=jnp.float32,
        )
        logs = jnp.log(jnp.where(par == 1, s0_ref[...], s1_ref[...]))
        out_ref[...] = x1 - logs


def _mm_log_softmax(wt_aug, se_aug):
    return pl.pallas_call(
        _mm_lse_kernel,
        grid=(NBB + 1, NV),
        in_specs=[
            pl.BlockSpec((D + 1, TV), lambda i, v: (0, v)),
            pl.BlockSpec((BB, D + 1), lambda i, v: (jnp.minimum(i, NBB - 1), 0)),
            pl.BlockSpec((BB, D + 1), lambda i, v: (jnp.maximum(i - 1, 0), 0)),
        ],
        out_specs=pl.BlockSpec(
            (TV, BB),
            lambda i, v: (jnp.where(i == 0, 0, v), jnp.maximum(i - 1, 0)),
        ),
        out_shape=jax.ShapeDtypeStruct((VOCAB, B), jnp.float32),
        scratch_shapes=[
            pltpu.VMEM((1, BB), jnp.float32),
            pltpu.VMEM((1, BB), jnp.float32),
        ],
        compiler_params=pltpu.CompilerParams(
            dimension_semantics=("arbitrary", "arbitrary"),
        ),
    )(wt_aug, se_aug, se_aug)


def kernel(context, emb_table, W, b):
    table_lin = _relayout_table(emb_table.T).reshape(TROWS, D)
    ctx = context.astype(jnp.int32)
    remapped = jnp.where(ctx < SPLIT, 2 * ctx, 2 * (ctx - SPLIT) + 1)
    idx = remapped.reshape(NW, NCHUNK, GCHUNK)
    sum_embeds = _gather_sum(idx, table_lin)

    wt = jnp.pad(W.T.astype(jnp.bfloat16), ((0, 0), (0, VP - VOCAB)))
    b_row = jnp.pad(b.astype(jnp.bfloat16), (0, VP - VOCAB),
                    constant_values=jnp.bfloat16(-30000.0))
    wt_aug = jnp.concatenate([wt, b_row[None, :]], axis=0)
    se_aug = jnp.concatenate(
        [sum_embeds.astype(jnp.bfloat16), jnp.ones((B, 1), jnp.bfloat16)],
        axis=1)

    out_t = _mm_log_softmax(wt_aug, se_aug)
    return out_t.T
